# Initial kernel scaffold; baseline (speedup 1.0000x reference)
import jax
import jax.numpy as jnp
from jax.experimental import pallas as pl


def kernel(x, sunlocked_lon_grid, params):
    raise NotImplementedError("write your pallas kernel here")


# speedup vs baseline: 2.5408x; 2.5408x over previous
import numpy as np
import jax
import jax.numpy as jnp
from jax.experimental import pallas as pl
from jax.experimental.pallas import tpu as pltpu

B, INC, H, W = 8, 4, 180, 360
C = 64
DEPTH = 6
ML, MW = 32, 64
NH = 360
OC = 2
EPS = 1e-5
HC = 45
NHC = H // HC

_PREC = jax.lax.Precision.HIGHEST
_CP = pltpu.CompilerParams(
    dimension_semantics=("parallel",),
    vmem_limit_bytes=56 * 1024 * 1024,
)
_CP2 = pltpu.CompilerParams(
    dimension_semantics=("parallel", "arbitrary"),
    vmem_limit_bytes=56 * 1024 * 1024,
)


def _build_mats():
    kh = np.arange(ML)[:, None].astype(np.float64)
    h = np.arange(H)[None, :].astype(np.float64)
    ang = 2.0 * np.pi * kh * h / H
    scale = 1.0 / np.sqrt(H * W)
    chf = (np.cos(ang) * scale)
    shf = (np.sin(ang) * scale)
    w = np.arange(W)[:, None].astype(np.float64)
    kw = np.arange(MW)[None, :].astype(np.float64)
    angw = 2.0 * np.pi * w * kw / W
    cwf = np.cos(angw)
    swf = np.sin(angw)
    h2 = np.arange(H)[:, None].astype(np.float64)
    kh2 = np.arange(ML)[None, :].astype(np.float64)
    ang2 = 2.0 * np.pi * h2 * kh2 / H
    chi = np.cos(ang2)
    shi = np.sin(ang2)
    kw2 = np.arange(MW)[:, None].astype(np.float64)
    w2 = np.arange(W)[None, :].astype(np.float64)
    ang3 = 2.0 * np.pi * kw2 * w2 / W
    alpha = np.where(np.arange(MW) == 0, 1.0, 2.0)[:, None] * scale
    cwi = np.cos(ang3) * alpha
    swi = np.sin(ang3) * alpha
    return [a.astype(np.float32) for a in (chf, shf, cwf, swf, chi, shi, cwi, swi)]


_CHF, _SHF, _CWF, _SWF, _CHI, _SHI, _CWI, _SWI = _build_mats()


def _dg(a, b, dn):
    return jax.lax.dot_general(a, b, dn, precision=_PREC,
                               preferred_element_type=jnp.float32)


def _gelu(t):
    return 0.5 * t * (1.0 + jax.lax.erf(t * np.float32(1.0 / np.sqrt(2.0))))


def _in_kernel(x_ref, w_ref, b_ref, o_ref):
    xb = x_ref[0]
    y = _dg(w_ref[...], xb, (((1,), (0,)), ((), ())))
    o_ref[0] = y + b_ref[...]


def _in_proj(x2, in_w, in_b2):
    return pl.pallas_call(
        _in_kernel,
        grid=(B,),
        in_specs=[
            pl.BlockSpec((1, INC, H * W), lambda b: (b, 0, 0)),
            pl.BlockSpec((C, INC), lambda b: (0, 0)),
            pl.BlockSpec((C, 1), lambda b: (0, 0)),
        ],
        out_specs=pl.BlockSpec((1, C, H * W), lambda b: (b, 0, 0)),
        out_shape=jax.ShapeDtypeStruct((B, C, H * W), jnp.float32),
        compiler_params=_CP,
    )(x2, in_w, in_b2)


CC = 16
NCC = C // CC


def _fwd_kernel(x_ref, chf_ref, shf_ref, cwf_ref, swf_ref, fr_ref, fi_ref):
    xb = x_ref[0]
    chb = jnp.broadcast_to(chf_ref[...][None], (CC, ML, H))
    shb = jnp.broadcast_to(shf_ref[...][None], (CC, ML, H))
    dn_b = (((2,), (1,)), ((0,), (0,)))
    u = _dg(chb, xb, dn_b)
    s = _dg(shb, xb, dn_b)
    dn_t = (((2,), (0,)), ((), ()))
    fr_ref[0] = _dg(u, cwf_ref[...], dn_t) - _dg(s, swf_ref[...], dn_t)
    fi_ref[0] = -(_dg(u, swf_ref[...], dn_t) + _dg(s, cwf_ref[...], dn_t))


def _fwd(x):
    return pl.pallas_call(
        _fwd_kernel,
        grid=(B, NCC),
        in_specs=[
            pl.BlockSpec((1, CC, H, W), lambda b, c: (b, c, 0, 0)),
            pl.BlockSpec((ML, H), lambda b, c: (0, 0)),
            pl.BlockSpec((ML, H), lambda b, c: (0, 0)),
            pl.BlockSpec((W, MW), lambda b, c: (0, 0)),
            pl.BlockSpec((W, MW), lambda b, c: (0, 0)),
        ],
        out_specs=[
            pl.BlockSpec((1, CC, ML, MW), lambda b, c: (b, c, 0, 0)),
            pl.BlockSpec((1, CC, ML, MW), lambda b, c: (b, c, 0, 0)),
        ],
        out_shape=[
            jax.ShapeDtypeStruct((B, C, ML, MW), jnp.float32),
            jax.ShapeDtypeStruct((B, C, ML, MW), jnp.float32),
        ],
        compiler_params=_CP2,
    )(x, jnp.asarray(_CHF), jnp.asarray(_SHF), jnp.asarray(_CWF), jnp.asarray(_SWF))


def _mix_kernel(fr_ref, fi_ref, wr_ref, wi_ref, gr_ref, gi_ref):
    fr = fr_ref[0]
    fi = fi_ref[0]
    wr = wr_ref[0]
    wi = wi_ref[0]
    dn = (((2,), (1,)), ((0,), (0,)))
    gr_ref[0] = _dg(fr, wr, dn) - _dg(fi, wi, dn)
    gi_ref[0] = _dg(fr, wi, dn) + _dg(fi, wr, dn)


def _mix(frt, fit, wrd, wid):
    return pl.pallas_call(
        _mix_kernel,
        grid=(ML,),
        in_specs=[
            pl.BlockSpec((1, MW, B, C), lambda k: (k, 0, 0, 0)),
            pl.BlockSpec((1, MW, B, C), lambda k: (k, 0, 0, 0)),
            pl.BlockSpec((1, MW, C, C), lambda k: (k, 0, 0, 0)),
            pl.BlockSpec((1, MW, C, C), lambda k: (k, 0, 0, 0)),
        ],
        out_specs=[
            pl.BlockSpec((1, MW, B, C), lambda k: (k, 0, 0, 0)),
            pl.BlockSpec((1, MW, B, C), lambda k: (k, 0, 0, 0)),
        ],
        out_shape=[
            jax.ShapeDtypeStruct((ML, MW, B, C), jnp.float32),
            jax.ShapeDtypeStruct((ML, MW, B, C), jnp.float32),
        ],
        compiler_params=_CP,
    )(frt, fit, wrd, wid)


def _inv_kernel(x_ref, gr_ref, gi_ref, chi_ref, shi_ref, cwi_ref, swi_ref,
                lw_ref, lb_ref, lg_ref, lbeta_ref, o_ref):
    gr = gr_ref[0]
    gi = gi_ref[0]
    chb = jnp.broadcast_to(chi_ref[0][None], (C, HC, ML))
    shb = jnp.broadcast_to(shi_ref[0][None], (C, HC, ML))
    dn_b = (((2,), (1,)), ((0,), (0,)))
    yr = _dg(chb, gr, dn_b) - _dg(shb, gi, dn_b)
    yi = _dg(chb, gi, dn_b) + _dg(shb, gr, dn_b)
    dn_t = (((2,), (0,)), ((), ()))
    x1 = _dg(yr, cwi_ref[...], dn_t) - _dg(yi, swi_ref[...], dn_t)
    xb = x_ref[0, :, 0]
    x2 = _dg(lw_ref[...], xb, (((1,), (0,)), ((), ())))
    t = _gelu(x1 + x2 + lb_ref[...][:, :, None])
    mu = jnp.mean(t, axis=0, keepdims=True)
    var = jnp.mean((t - mu) ** 2, axis=0, keepdims=True)
    o = (t - mu) * jax.lax.rsqrt(var + EPS)
    o = o * lg_ref[...][:, :, None] + lbeta_ref[...][:, :, None]
    o_ref[0, :, 0] = _gelu(o)


def _inv(x, gr, gi, lw, lb2, lg2, lbeta2):
    x5 = x.reshape(B, C, NHC, HC, W)
    chi5 = jnp.asarray(_CHI).reshape(NHC, HC, ML)
    shi5 = jnp.asarray(_SHI).reshape(NHC, HC, ML)
    out = pl.pallas_call(
        _inv_kernel,
        grid=(B, NHC),
        in_specs=[
            pl.BlockSpec((1, C, 1, HC, W), lambda b, j: (b, 0, j, 0, 0)),
            pl.BlockSpec((1, C, ML, MW), lambda b, j: (b, 0, 0, 0)),
            pl.BlockSpec((1, C, ML, MW), lambda b, j: (b, 0, 0, 0)),
            pl.BlockSpec((1, HC, ML), lambda b, j: (j, 0, 0)),
            pl.BlockSpec((1, HC, ML), lambda b, j: (j, 0, 0)),
            pl.BlockSpec((MW, W), lambda b, j: (0, 0)),
            pl.BlockSpec((MW, W), lambda b, j: (0, 0)),
            pl.BlockSpec((C, C), lambda b, j: (0, 0)),
            pl.BlockSpec((C, 1), lambda b, j: (0, 0)),
            pl.BlockSpec((C, 1), lambda b, j: (0, 0)),
            pl.BlockSpec((C, 1), lambda b, j: (0, 0)),
        ],
        out_specs=pl.BlockSpec((1, C, 1, HC, W), lambda b, j: (b, 0, j, 0, 0)),
        out_shape=jax.ShapeDtypeStruct((B, C, NHC, HC, W), jnp.float32),
        compiler_params=_CP2,
    )(x5, gr, gi, chi5, shi5, jnp.asarray(_CWI),
      jnp.asarray(_SWI), lw, lb2, lg2, lbeta2)
    return out.reshape(B, C, H, W)


def _head_kernel(xt_ref, idx_ref, thi_ref, tlo_ref, mu_ref, lv_ref):
    idx = idx_ref[0, 0]
    iota = jax.lax.broadcasted_iota(jnp.int32, (HC, W, NH), 2)
    oh = (idx[:, :, None] == iota).astype(jnp.bfloat16)
    dn_t = (((2,), (0,)), ((), ()))
    g = (jax.lax.dot_general(oh, thi_ref[...], dn_t,
                             preferred_element_type=jnp.float32)
         + jax.lax.dot_general(oh, tlo_ref[...], dn_t,
                               preferred_element_type=jnp.float32))
    xc = xt_ref[0, 0]
    mu_ref[0, 0] = jnp.sum(g[:, :, 0:C] * xc, axis=-1) + g[:, :, 2 * C]
    lv_ref[0, 0] = jnp.sum(g[:, :, C:2 * C] * xc, axis=-1) + g[:, :, 2 * C + 1]


def _head(xt, idx, thi, tlo):
    xt5 = xt.reshape(B, NHC, HC, W, C)
    idx4 = idx.reshape(B, NHC, HC, W)
    mu, lv = pl.pallas_call(
        _head_kernel,
        grid=(B, NHC),
        in_specs=[
            pl.BlockSpec((1, 1, HC, W, C), lambda b, j: (b, j, 0, 0, 0)),
            pl.BlockSpec((1, 1, HC, W), lambda b, j: (b, j, 0, 0)),
            pl.BlockSpec((NH, 2 * C + 2), lambda b, j: (0, 0)),
            pl.BlockSpec((NH, 2 * C + 2), lambda b, j: (0, 0)),
        ],
        out_specs=[
            pl.BlockSpec((1, 1, HC, W), lambda b, j: (b, j, 0, 0)),
            pl.BlockSpec((1, 1, HC, W), lambda b, j: (b, j, 0, 0)),
        ],
        out_shape=[
            jax.ShapeDtypeStruct((B, NHC, HC, W), jnp.float32),
            jax.ShapeDtypeStruct((B, NHC, HC, W), jnp.float32),
        ],
        compiler_params=_CP2,
    )(xt5, idx4, thi, tlo)
    return mu.reshape(B, H, W), lv.reshape(B, H, W)


def kernel(x, sunlocked_lon_grid, params):
    x0 = _in_proj(x.reshape(B, INC, H * W), params['in_w'],
                  params['in_b'][:, None]).reshape(B, C, H, W)
    wr_all = jnp.transpose(params['four_w'][..., 0], (0, 3, 4, 1, 2))
    wi_all = jnp.transpose(params['four_w'][..., 1], (0, 3, 4, 1, 2))
    xd = x0
    for d in range(DEPTH):
        fr, fi = _fwd(xd)
        frt = jnp.transpose(fr, (2, 3, 0, 1))
        fit = jnp.transpose(fi, (2, 3, 0, 1))
        grt, git = _mix(frt, fit, wr_all[d], wi_all[d])
        gr = jnp.transpose(grt, (2, 3, 0, 1))
        gi = jnp.transpose(git, (2, 3, 0, 1))
        xd = _inv(xd, gr, gi, params['lin_w'][d],
                  params['lin_b'][d][:, None], params['ln_g'][d][:, None],
                  params['ln_b'][d][:, None])
    xt = jnp.transpose(xd, (0, 2, 3, 1))
    idx = jnp.clip(sunlocked_lon_grid, 0, NH - 1).astype(jnp.int32)
    table = jnp.concatenate(
        [params['head_w'].reshape(NH, OC * C), params['head_b']], axis=1)
    thi = table.astype(jnp.bfloat16)
    tlo = (table - thi.astype(jnp.float32)).astype(jnp.bfloat16)
    mu, lv = _head(xt, idx, thi, tlo)
    return mu.reshape(B, 1, H, W), lv.reshape(B, 1, H, W)
